# Initial kernel scaffold; baseline (speedup 1.0000x reference)
import functools

import jax
import jax.numpy as jnp
from jax import lax
from jax.experimental import pallas as pl
from jax.experimental.pallas import tpu as pltpu
from jax.experimental.pallas import tpu_sc as plsc

NUM_EMB = 1000000
D = 32
N = 4096 * 200

NC = 2
NS = 16
NW = NC * NS
R = N // NW

GRP = 128
CHUNK = 1280
G_PER_CHUNK = CHUNK // GRP
NCHUNK = R // CHUNK


@functools.partial(
    pl.kernel,
    out_type=jax.ShapeDtypeStruct((N, D), jnp.float32),
    mesh=plsc.VectorSubcoreMesh(core_axis_name="c", subcore_axis_name="s"),
    scratch_types=[
        pltpu.VMEM((R,), jnp.int32),
        pltpu.VMEM((CHUNK, D), jnp.float32),
        pltpu.SemaphoreType.DMA,
    ],
)
def _sc_gather(idx_hbm, table_hbm, out_hbm, idx_v, rows_v, sem):
    wid = lax.axis_index("s") * NC + lax.axis_index("c")
    base = wid * R
    pltpu.sync_copy(idx_hbm.at[pl.ds(base, R)], idx_v)

    @pl.loop(0, NCHUNK)
    def _chunk(c):
        coff = c * CHUNK
        copies = []
        for g in range(G_PER_CHUNK):
            copies.append(pltpu.async_copy(
                table_hbm.at[idx_v.at[pl.ds(coff + g * GRP, GRP)]],
                rows_v.at[pl.ds(g * GRP, GRP)],
                sem,
            ))
        for cp in copies:
            cp.wait()
        pltpu.sync_copy(rows_v, out_hbm.at[pl.ds(base + coff, CHUNK)])


def kernel(x, weights):
    idx = x.reshape(-1)
    out = _sc_gather(idx, weights)
    return out.reshape(x.shape + (weights.shape[1],))


# speedup vs baseline: 1.4832x; 1.4832x over previous
import functools

import jax
import jax.numpy as jnp
from jax import lax
from jax.experimental import pallas as pl
from jax.experimental.pallas import tpu as pltpu
from jax.experimental.pallas import tpu_sc as plsc

NUM_EMB = 1000000
D = 32
N = 4096 * 200

NC = 2
NS = 16
NW = NC * NS
R = N // NW

GRP = 128
CHUNK = 1280
G_PER_CHUNK = CHUNK // GRP
NCHUNK = R // CHUNK


@functools.partial(
    pl.kernel,
    out_type=jax.ShapeDtypeStruct((N, D), jnp.float32),
    mesh=plsc.VectorSubcoreMesh(core_axis_name="c", subcore_axis_name="s"),
    scratch_types=[
        pltpu.VMEM((R,), jnp.int32),
        pltpu.VMEM((CHUNK, D), jnp.float32),
        pltpu.SemaphoreType.DMA,
    ],
    compiler_params=pltpu.CompilerParams(use_tc_tiling_on_sc=False),
)
def _sc_gather(idx_hbm, table_hbm, out_hbm, idx_v, rows_v, sem):
    wid = lax.axis_index("s") * NC + lax.axis_index("c")
    base = wid * R
    pltpu.sync_copy(idx_hbm.at[pl.ds(base, R)], idx_v)

    @pl.loop(0, NCHUNK)
    def _chunk(c):
        coff = c * CHUNK
        copies = []
        for g in range(G_PER_CHUNK):
            copies.append(pltpu.async_copy(
                table_hbm.at[idx_v.at[pl.ds(coff + g * GRP, GRP)]],
                rows_v.at[pl.ds(g * GRP, GRP)],
                sem,
            ))
        for cp in copies:
            cp.wait()
        pltpu.sync_copy(rows_v, out_hbm.at[pl.ds(base + coff, CHUNK)])


def kernel(x, weights):
    idx = x.reshape(-1)
    out = _sc_gather(idx, weights)
    return out.reshape(x.shape + (weights.shape[1],))


# speedup vs baseline: 1.4998x; 1.0112x over previous
import functools

import jax
import jax.numpy as jnp
from jax import lax
from jax.experimental import pallas as pl
from jax.experimental.pallas import tpu as pltpu
from jax.experimental.pallas import tpu_sc as plsc

NUM_EMB = 1000000
D = 32
N = 4096 * 200

NC = 2
NS = 16
NW = NC * NS
R = N // NW

GRP = 128
CHUNK = 1280
G_PER_CHUNK = CHUNK // GRP
NCHUNK = R // CHUNK


@functools.partial(
    pl.kernel,
    out_type=jax.ShapeDtypeStruct((N, D), jnp.float32),
    mesh=plsc.VectorSubcoreMesh(core_axis_name="c", subcore_axis_name="s"),
    scratch_types=[
        pltpu.VMEM((R,), jnp.int32),
        pltpu.VMEM((CHUNK, D), jnp.float32),
        pltpu.VMEM((CHUNK, D), jnp.float32),
        pltpu.SemaphoreType.DMA,
        pltpu.SemaphoreType.DMA,
        pltpu.SemaphoreType.DMA,
        pltpu.SemaphoreType.DMA,
    ],
    compiler_params=pltpu.CompilerParams(use_tc_tiling_on_sc=False),
)
def _sc_gather(idx_hbm, table_hbm, out_hbm, idx_v, rows0, rows1,
               sg0, sg1, so0, so1):
    wid = lax.axis_index("s") * NC + lax.axis_index("c")
    base = wid * R
    pltpu.sync_copy(idx_hbm.at[pl.ds(base, R)], idx_v)

    rows = (rows0, rows1)
    sg = (sg0, sg1)
    so = (so0, so1)

    def fire(c, b):
        coff = c * CHUNK
        for g in range(G_PER_CHUNK):
            pltpu.async_copy(
                table_hbm.at[idx_v.at[pl.ds(coff + g * GRP, GRP)]],
                rows[b].at[pl.ds(g * GRP, GRP)],
                sg[b],
            )

    def drain_gathers(b):
        for g in range(G_PER_CHUNK):
            pltpu.make_async_copy(
                out_hbm.at[pl.ds(base, GRP)],
                rows[b].at[pl.ds(g * GRP, GRP)],
                sg[b],
            ).wait()

    def wait_outcopy(b):
        pltpu.make_async_copy(
            rows[b],
            out_hbm.at[pl.ds(base, CHUNK)],
            so[b],
        ).wait()

    fire(0, 0)

    @pl.loop(0, NCHUNK // 2)
    def _pair(i):
        for b in range(2):
            c = i * 2 + b
            nb = 1 - b

            @pl.when(c + 1 < NCHUNK)
            def _fire_next():
                @pl.when(c + 1 >= 2)
                def _buffer_free():
                    wait_outcopy(nb)
                fire(c + 1, nb)

            drain_gathers(b)
            pltpu.async_copy(
                rows[b],
                out_hbm.at[pl.ds(base + c * CHUNK, CHUNK)],
                so[b],
            )

    wait_outcopy(0)
    wait_outcopy(1)


def kernel(x, weights):
    idx = x.reshape(-1)
    out = _sc_gather(idx, weights)
    return out.reshape(x.shape + (weights.shape[1],))


# speedup vs baseline: 2.0495x; 1.3665x over previous
import functools

import jax
import jax.numpy as jnp
from jax import lax
from jax.experimental import pallas as pl
from jax.experimental.pallas import tpu as pltpu
from jax.experimental.pallas import tpu_sc as plsc
from jax.experimental import layout as jlayout

NUM_EMB = 1000000
D = 32
N = 4096 * 200
OUT_W = 128

NC = 2
NS = 16
NW = NC * NS
R = N // NW

GRP = 128
CHUNK = 1280
G_PER_CHUNK = CHUNK // GRP
NCHUNK = R // CHUNK


@functools.partial(
    pl.kernel,
    out_type=jax.ShapeDtypeStruct((N, OUT_W), jnp.float32),
    mesh=plsc.VectorSubcoreMesh(core_axis_name="c", subcore_axis_name="s"),
    scratch_types=[
        pltpu.VMEM((R,), jnp.int32),
        pltpu.VMEM((CHUNK, D), jnp.float32),
        pltpu.VMEM((CHUNK, D), jnp.float32),
        pltpu.SemaphoreType.DMA,
        pltpu.SemaphoreType.DMA,
        pltpu.SemaphoreType.DMA,
        pltpu.SemaphoreType.DMA,
    ],
    compiler_params=pltpu.CompilerParams(use_tc_tiling_on_sc=False),
)
def _sc_gather(idx_hbm, table_hbm, out_hbm, idx_v, rows0, rows1,
               sg0, sg1, so0, so1):
    wid = lax.axis_index("s") * NC + lax.axis_index("c")
    base = wid * R
    pltpu.sync_copy(idx_hbm.at[pl.ds(base, R)], idx_v)

    rows = (rows0, rows1)
    sg = (sg0, sg1)
    so = (so0, so1)

    def fire(c, b):
        coff = c * CHUNK
        for g in range(G_PER_CHUNK):
            pltpu.async_copy(
                table_hbm.at[idx_v.at[pl.ds(coff + g * GRP, GRP)]],
                rows[b].at[pl.ds(g * GRP, GRP)],
                sg[b],
            )

    def drain_gathers(b):
        for g in range(G_PER_CHUNK):
            pltpu.make_async_copy(
                out_hbm.at[pl.ds(base, GRP), pl.ds(0, D)],
                rows[b].at[pl.ds(g * GRP, GRP)],
                sg[b],
            ).wait()

    def start_outcopy(b, c):
        pltpu.async_copy(
            rows[b],
            out_hbm.at[pl.ds(base + c * CHUNK, CHUNK), pl.ds(0, D)],
            so[b],
        )

    def wait_outcopy(b):
        pltpu.make_async_copy(
            rows[b],
            out_hbm.at[pl.ds(base, CHUNK), pl.ds(0, D)],
            so[b],
        ).wait()

    fire(0, 0)

    @pl.loop(0, NCHUNK // 2)
    def _pair(i):
        for b in range(2):
            c = i * 2 + b
            nb = 1 - b

            @pl.when(c + 1 < NCHUNK)
            def _fire_next():
                @pl.when(c + 1 >= 2)
                def _buffer_free():
                    wait_outcopy(nb)
                fire(c + 1, nb)

            drain_gathers(b)
            start_outcopy(b, c)

    wait_outcopy(0)
    wait_outcopy(1)


def kernel(x, weights):
    idx = x.reshape(-1)
    out = _sc_gather(idx, weights)
    out = out[:, :D].reshape(x.shape + (weights.shape[1],))
    return out


# speedup vs baseline: 2.8076x; 1.3699x over previous
import functools

import jax
import jax.numpy as jnp
from jax import lax
from jax.experimental import pallas as pl
from jax.experimental.pallas import tpu as pltpu
from jax.experimental.pallas import tpu_sc as plsc
from jax.experimental import layout as jlayout

NUM_EMB = 1000000
D = 32
N = 4096 * 200
OUT_W = 128

NC = 2
NS = 16
NW = NC * NS
R = N // NW

GRP = 128
CHUNK = 1280
G_PER_CHUNK = CHUNK // GRP
NCHUNK = R // CHUNK


@functools.partial(
    pl.kernel,
    out_type=jax.ShapeDtypeStruct((N, OUT_W), jnp.float32),
    mesh=plsc.VectorSubcoreMesh(core_axis_name="c", subcore_axis_name="s"),
    scratch_types=[
        pltpu.VMEM((R,), jnp.int32),
        pltpu.VMEM((CHUNK, D), jnp.float32),
        pltpu.VMEM((CHUNK, D), jnp.float32),
        pltpu.SemaphoreType.DMA,
        pltpu.SemaphoreType.DMA,
        pltpu.SemaphoreType.DMA,
        pltpu.SemaphoreType.DMA,
    ],
    compiler_params=pltpu.CompilerParams(use_tc_tiling_on_sc=False),
)
def _sc_gather(idx_hbm, table_hbm, out_hbm, idx_v, rows0, rows1,
               sg0, sg1, so0, so1):
    wid = lax.axis_index("s") * NC + lax.axis_index("c")
    base = wid * R
    pltpu.sync_copy(idx_hbm.at[pl.ds(base, R)], idx_v)

    rows = (rows0, rows1)
    sg = (sg0, sg1)
    so = (so0, so1)

    def fire(c, b):
        coff = c * CHUNK
        for g in range(G_PER_CHUNK):
            pltpu.async_copy(
                table_hbm.at[idx_v.at[pl.ds(coff + g * GRP, GRP)]],
                rows[b].at[pl.ds(g * GRP, GRP)],
                sg[b],
            )

    def drain_gathers(b):
        for g in range(G_PER_CHUNK):
            pltpu.make_async_copy(
                out_hbm.at[pl.ds(base, GRP), pl.ds(0, D)],
                rows[b].at[pl.ds(g * GRP, GRP)],
                sg[b],
            ).wait()

    def start_outcopy(b, c):
        pltpu.async_copy(
            rows[b],
            out_hbm.at[pl.ds(base + c * CHUNK, CHUNK), pl.ds(0, D)],
            so[b],
        )

    def wait_outcopy(b):
        pltpu.make_async_copy(
            rows[b],
            out_hbm.at[pl.ds(base, CHUNK), pl.ds(0, D)],
            so[b],
        ).wait()

    fire(0, 0)

    @pl.loop(0, NCHUNK // 2)
    def _pair(i):
        for b in range(2):
            c = i * 2 + b
            nb = 1 - b

            @pl.when(c + 1 < NCHUNK)
            def _fire_next():
                @pl.when(c + 1 >= 2)
                def _buffer_free():
                    wait_outcopy(nb)
                fire(c + 1, nb)

            drain_gathers(b)
            start_outcopy(b, c)

    wait_outcopy(0)
    wait_outcopy(1)


def kernel(x, weights):
    idx = x.reshape(-1)
    wlin = jlayout.with_layout_constraint(
        weights,
        jlayout.Layout(major_to_minor=(0, 1), tiling=((32, 32),)),
    )
    out = _sc_gather(idx, wlin)
    out = out[:, :D].reshape(x.shape + (weights.shape[1],))
    return out
